# baseline (speedup 1.0000x reference)
import functools

import jax
import jax.numpy as jnp
from jax import lax
from jax.experimental import pallas as pl
from jax.experimental.pallas import tpu as pltpu
from jax.experimental.pallas import tpu_sc as plsc

_NUM_CORES = 2
_NUM_SUBCORES = 16
_NW = _NUM_CORES * _NUM_SUBCORES


def _make_gather(batch: int, hist: int, dim: int, chunk: int, nbuf: int):
    n_rows = batch * hist
    assert n_rows % _NW == 0
    per_w = n_rows // _NW
    assert per_w % chunk == 0
    n_chunks = per_w // chunk

    mesh = plsc.VectorSubcoreMesh(core_axis_name="c", subcore_axis_name="s")

    @functools.partial(
        pl.kernel,
        out_type=jax.ShapeDtypeStruct((n_rows, dim), jnp.float32),
        mesh=mesh,
        scratch_types=[
            pltpu.VMEM((per_w,), jnp.int32),
            [pltpu.VMEM((chunk, dim), jnp.float32) for _ in range(nbuf)],
            [pltpu.SemaphoreType.DMA for _ in range(nbuf)],
            [pltpu.SemaphoreType.DMA for _ in range(nbuf)],
        ],
    )
    def gather_kernel(idx_hbm, table_hbm, out_hbm, idx_v, rows, gsem, ssem):
        wid = lax.axis_index("s") * _NUM_CORES + lax.axis_index("c")
        base = wid * per_w
        out_flat = out_hbm

        pltpu.sync_copy(idx_hbm.at[pl.ds(base, per_w)], idx_v)

        def gather_cp(c):
            b = c % nbuf
            return pltpu.make_async_copy(
                table_hbm.at[idx_v.at[pl.ds(c * chunk, chunk)]], rows[b], gsem[b]
            )

        def scatter_cp(c):
            b = c % nbuf
            return pltpu.make_async_copy(
                rows[b], out_flat.at[pl.ds(base + c * chunk, chunk)], ssem[b]
            )

        ahead = max(1, nbuf - 1)
        waited = set()
        for c in range(min(ahead, n_chunks)):
            gather_cp(c).start()
        for c in range(n_chunks):
            gather_cp(c).wait()
            scatter_cp(c).start()
            nxt = c + ahead
            if nxt < n_chunks:
                prev = nxt - nbuf
                if prev >= 0:
                    scatter_cp(prev).wait()
                    waited.add(prev)
                gather_cp(nxt).start()
        for c in range(n_chunks):
            if c not in waited:
                scatter_cp(c).wait()

    return gather_kernel


def kernel(prompt_ids, prompts):
    batch, hist = prompt_ids.shape
    _, dim = prompts.shape
    n_rows = batch * hist
    idx = prompt_ids.T.reshape(n_rows).astype(jnp.int32)
    out = _make_gather(batch, hist, dim, chunk=320, nbuf=3)(idx, prompts)
    return out.reshape(hist, batch, dim).transpose(1, 0, 2)


# speedup vs baseline: 1.0121x; 1.0121x over previous
import functools

import jax
import jax.numpy as jnp
from jax import lax
from jax.experimental import pallas as pl
from jax.experimental.pallas import tpu as pltpu
from jax.experimental.pallas import tpu_sc as plsc

_NUM_CORES = 2
_NUM_SUBCORES = 16
_NW = _NUM_CORES * _NUM_SUBCORES


def _make_gather(batch: int, hist: int, dim: int, chunk: int, nbuf: int):
    n_rows = batch * hist
    assert n_rows % _NW == 0
    per_w = n_rows // _NW
    assert per_w % chunk == 0
    n_chunks = per_w // chunk

    mesh = plsc.VectorSubcoreMesh(core_axis_name="c", subcore_axis_name="s")

    @functools.partial(
        pl.kernel,
        out_type=jax.ShapeDtypeStruct((n_rows, dim), jnp.float32),
        mesh=mesh,
        scratch_types=[
            pltpu.VMEM((per_w,), jnp.int32),
            [pltpu.VMEM((chunk, dim), jnp.float32) for _ in range(nbuf)],
            [pltpu.SemaphoreType.DMA for _ in range(nbuf)],
            [pltpu.SemaphoreType.DMA for _ in range(nbuf)],
        ],
    )
    def gather_kernel(idx_hbm, table_hbm, out_hbm, idx_v, rows, gsem, ssem):
        wid = lax.axis_index("s") * _NUM_CORES + lax.axis_index("c")
        base = wid * per_w
        out_flat = out_hbm

        pltpu.sync_copy(idx_hbm.at[pl.ds(base, per_w)], idx_v)

        def gather_cp(c):
            b = c % nbuf
            return pltpu.make_async_copy(
                table_hbm.at[idx_v.at[pl.ds(c * chunk, chunk)]], rows[b], gsem[b]
            )

        def scatter_cp(c):
            b = c % nbuf
            return pltpu.make_async_copy(
                rows[b], out_flat.at[pl.ds(base + c * chunk, chunk)], ssem[b]
            )

        ahead = max(1, nbuf - 1)
        waited = set()
        for c in range(min(ahead, n_chunks)):
            gather_cp(c).start()
        for c in range(n_chunks):
            gather_cp(c).wait()
            scatter_cp(c).start()
            nxt = c + ahead
            if nxt < n_chunks:
                prev = nxt - nbuf
                if prev >= 0:
                    scatter_cp(prev).wait()
                    waited.add(prev)
                gather_cp(nxt).start()
        for c in range(n_chunks):
            if c not in waited:
                scatter_cp(c).wait()

    return gather_kernel


def kernel(prompt_ids, prompts):
    batch, hist = prompt_ids.shape
    _, dim = prompts.shape
    n_rows = batch * hist
    idx = prompt_ids.T.reshape(n_rows).astype(jnp.int32)
    out = _make_gather(batch, hist, dim, chunk=128, nbuf=4)(idx, prompts)
    return out.reshape(hist, batch, dim).transpose(1, 0, 2)


# speedup vs baseline: 1.0311x; 1.0188x over previous
import functools

import jax
import jax.numpy as jnp
from jax import lax
from jax.experimental import pallas as pl
from jax.experimental.pallas import tpu as pltpu
from jax.experimental.pallas import tpu_sc as plsc

_NUM_CORES = 2
_NUM_SUBCORES = 16
_NW = _NUM_CORES * _NUM_SUBCORES


def _make_gather(batch: int, hist: int, dim: int, chunk: int, nbuf: int):
    n_rows = batch * hist
    assert n_rows % _NW == 0
    per_w = n_rows // _NW
    assert per_w % chunk == 0
    n_chunks = per_w // chunk

    mesh = plsc.VectorSubcoreMesh(core_axis_name="c", subcore_axis_name="s")

    @functools.partial(
        pl.kernel,
        out_type=jax.ShapeDtypeStruct((n_rows, dim), jnp.float32),
        mesh=mesh,
        scratch_types=[
            pltpu.VMEM((per_w,), jnp.int32),
            [pltpu.VMEM((chunk, dim), jnp.float32) for _ in range(nbuf)],
            [pltpu.SemaphoreType.DMA for _ in range(nbuf)],
            [pltpu.SemaphoreType.DMA for _ in range(nbuf)],
        ],
    )
    def gather_kernel(idx_hbm, table_hbm, out_hbm, idx_v, rows, gsem, ssem):
        wid = lax.axis_index("s") * _NUM_CORES + lax.axis_index("c")
        base = wid * per_w
        out_flat = out_hbm

        pltpu.sync_copy(idx_hbm.at[pl.ds(base, per_w)], idx_v)

        def gather_cp(c):
            b = c % nbuf
            return pltpu.make_async_copy(
                table_hbm.at[idx_v.at[pl.ds(c * chunk, chunk)]], rows[b], gsem[b]
            )

        def scatter_cp(c):
            b = c % nbuf
            return pltpu.make_async_copy(
                rows[b], out_flat.at[pl.ds(base + c * chunk, chunk)], ssem[b]
            )

        ahead = max(1, nbuf - 1)
        waited = set()
        for c in range(min(ahead, n_chunks)):
            gather_cp(c).start()
        for c in range(n_chunks):
            gather_cp(c).wait()
            scatter_cp(c).start()
            nxt = c + ahead
            if nxt < n_chunks:
                prev = nxt - nbuf
                if prev >= 0:
                    scatter_cp(prev).wait()
                    waited.add(prev)
                gather_cp(nxt).start()
        for c in range(n_chunks):
            if c not in waited:
                scatter_cp(c).wait()

    return gather_kernel


def kernel(prompt_ids, prompts):
    batch, hist = prompt_ids.shape
    _, dim = prompts.shape
    n_rows = batch * hist
    idx = prompt_ids.T.reshape(n_rows).astype(jnp.int32)
    out = _make_gather(batch, hist, dim, chunk=128, nbuf=6)(idx, prompts)
    return out.reshape(hist, batch, dim).transpose(1, 0, 2)
